# baseline (speedup 1.0000x reference)
import jax
import jax.numpy as jnp
from jax import lax
from jax.experimental import pallas as pl
from jax.experimental.pallas import tpu as pltpu
from jax.experimental.pallas import tpu_sc as plsc

NC = 2
NS = 16
NW = NC * NS
L = 16

ROWS = 524288
COLS = 8
TILE_ROWS = 128
TILES = ROWS // TILE_ROWS
TPW = TILES // NW
CHUNK_T = 16
NCHUNK = TPW // CHUNK_T
NBUF = 2

_mesh = plsc.VectorSubcoreMesh(core_axis_name="c", subcore_axis_name="s",
                               num_cores=NC, num_subcores=NS)


def _softmax_chunk(in_buf, out_buf):
    zeros = jnp.zeros((L,), jnp.float32)

    @plsc.parallel_loop(0, CHUNK_T, 1, unroll=2)
    def _tile(t):
        for k in range(TILE_ROWS // L):
            l0 = k * L
            cols = [in_buf[t, c, pl.ds(l0, L)] for c in range(7)]
            m01 = jnp.maximum(cols[0], cols[1])
            m23 = jnp.maximum(cols[2], cols[3])
            m45 = jnp.maximum(cols[4], cols[5])
            m = jnp.maximum(jnp.maximum(m01, m23),
                            jnp.maximum(m45, cols[6]))
            es = [jnp.exp(v - m) for v in cols]
            s01 = es[0] + es[1]
            s23 = es[2] + es[3]
            s45 = es[4] + es[5]
            s = (s01 + s23) + (s45 + es[6])
            inv = jnp.float32(1.0 / 256.0) / s
            for c in range(7):
                out_buf[t, c, pl.ds(l0, L)] = es[c] * inv
            out_buf[t, 7, pl.ds(l0, L)] = zeros


def _body(in_hbm, out_hbm,
          in_v0, in_v1, out_v0, out_v1, isem0, isem1, osem0, osem1):
    in_bufs = (in_v0, in_v1)
    out_bufs = (out_v0, out_v1)
    isems = (isem0, isem1)
    osems = (osem0, osem1)

    wid = lax.axis_index("s") * NC + lax.axis_index("c")
    base = wid * TPW

    for b in range(NBUF):
        pltpu.async_copy(
            in_hbm.at[pl.ds(base + b * CHUNK_T, CHUNK_T), pl.ds(0, 7)],
            in_bufs[b], isems[b])

    @pl.loop(0, NCHUNK, step=NBUF)
    def _outer(g0):
        for b in range(NBUF):
            g = g0 + b
            off = base + g * CHUNK_T
            pltpu.make_async_copy(
                in_hbm.at[pl.ds(off, CHUNK_T), pl.ds(0, 7)],
                in_bufs[b], isems[b]).wait()

            @pl.when(g0 > 0)
            def _():
                pltpu.make_async_copy(
                    out_bufs[b],
                    out_hbm.at[pl.ds(off - NBUF * CHUNK_T, CHUNK_T)],
                    osems[b]).wait()

            _softmax_chunk(in_bufs[b], out_bufs[b])

            pltpu.async_copy(out_bufs[b], out_hbm.at[pl.ds(off, CHUNK_T)],
                             osems[b])

            @pl.when(g + NBUF < NCHUNK)
            def _():
                pltpu.async_copy(
                    in_hbm.at[pl.ds(off + NBUF * CHUNK_T, CHUNK_T),
                              pl.ds(0, 7)],
                    in_bufs[b], isems[b])

    for b in range(NBUF):
        off = base + (NCHUNK - NBUF + b) * CHUNK_T
        pltpu.make_async_copy(out_bufs[b],
                              out_hbm.at[pl.ds(off, CHUNK_T)],
                              osems[b]).wait()


_sc_call = pl.kernel(
    _body,
    out_type=jax.ShapeDtypeStruct((TILES, COLS, TILE_ROWS), jnp.float32),
    mesh=_mesh,
    scratch_types=[
        pltpu.VMEM((CHUNK_T, COLS - 1, TILE_ROWS), jnp.float32),
        pltpu.VMEM((CHUNK_T, COLS - 1, TILE_ROWS), jnp.float32),
        pltpu.VMEM((CHUNK_T, COLS, TILE_ROWS), jnp.float32),
        pltpu.VMEM((CHUNK_T, COLS, TILE_ROWS), jnp.float32),
        pltpu.SemaphoreType.DMA,
        pltpu.SemaphoreType.DMA,
        pltpu.SemaphoreType.DMA,
        pltpu.SemaphoreType.DMA,
    ],
    compiler_params=pltpu.CompilerParams(needs_layout_passes=False,
                                         use_tc_tiling_on_sc=False),
)


def kernel(input):
    x3 = input.reshape(TILES, TILE_ROWS, COLS).swapaxes(1, 2)
    out3 = _sc_call(x3)
    return out3.swapaxes(1, 2).reshape(ROWS, COLS)


# speedup vs baseline: 1.6646x; 1.6646x over previous
import jax
import jax.numpy as jnp
from jax import lax
from jax.experimental import pallas as pl
from jax.experimental.pallas import tpu as pltpu
from jax.experimental.pallas import tpu_sc as plsc

NC = 2
NS = 16
NW = NC * NS
L = 16

ROWS = 524288
COLS = 8
TILE_ROWS = 128
TILES = ROWS // TILE_ROWS
TPW = TILES // NW
CHUNK_T = 16
NCHUNK = TPW // CHUNK_T
NBUF = 4

_mesh = plsc.VectorSubcoreMesh(core_axis_name="c", subcore_axis_name="s",
                               num_cores=NC, num_subcores=NS)


def _softmax_chunk(in_buf, out_buf):
    zeros = jnp.zeros((L,), jnp.float32)

    @plsc.parallel_loop(0, CHUNK_T * (TILE_ROWS // L), 1, unroll=4)
    def _grp(g):
        t = lax.shift_right_logical(g, 3)
        l0 = lax.shift_left(lax.bitwise_and(g, 7), 4)
        cols = [in_buf[t, c, pl.ds(l0, L)] for c in range(7)]
        m01 = jnp.maximum(cols[0], cols[1])
        m23 = jnp.maximum(cols[2], cols[3])
        m45 = jnp.maximum(cols[4], cols[5])
        m = jnp.maximum(jnp.maximum(m01, m23), jnp.maximum(m45, cols[6]))
        es = [jnp.exp(v - m) for v in cols]
        s01 = es[0] + es[1]
        s23 = es[2] + es[3]
        s45 = es[4] + es[5]
        s = (s01 + s23) + (s45 + es[6])
        inv = jnp.float32(1.0 / 256.0) / s
        for c in range(7):
            out_buf[t, c, pl.ds(l0, L)] = es[c] * inv
        out_buf[t, 7, pl.ds(l0, L)] = zeros


def _body(in_hbm, out_hbm, *scratch):
    in_bufs = scratch[0:NBUF]
    out_bufs = scratch[NBUF:2 * NBUF]
    isems = scratch[2 * NBUF:3 * NBUF]
    osems = scratch[3 * NBUF:4 * NBUF]

    wid = lax.axis_index("s") * NC + lax.axis_index("c")
    base = wid * TPW

    for b in range(NBUF):
        pltpu.async_copy(
            in_hbm.at[pl.ds(base + b * CHUNK_T, CHUNK_T), pl.ds(0, 7)],
            in_bufs[b], isems[b])

    @pl.loop(0, NCHUNK, step=NBUF)
    def _outer(g0):
        for b in range(NBUF):
            g = g0 + b
            off = base + g * CHUNK_T
            pltpu.make_async_copy(
                in_hbm.at[pl.ds(off, CHUNK_T), pl.ds(0, 7)],
                in_bufs[b], isems[b]).wait()

            @pl.when(g0 > 0)
            def _():
                pltpu.make_async_copy(
                    out_bufs[b],
                    out_hbm.at[pl.ds(off - NBUF * CHUNK_T, CHUNK_T)],
                    osems[b]).wait()

            _softmax_chunk(in_bufs[b], out_bufs[b])

            pltpu.async_copy(out_bufs[b], out_hbm.at[pl.ds(off, CHUNK_T)],
                             osems[b])

            @pl.when(g + NBUF < NCHUNK)
            def _():
                pltpu.async_copy(
                    in_hbm.at[pl.ds(off + NBUF * CHUNK_T, CHUNK_T),
                              pl.ds(0, 7)],
                    in_bufs[b], isems[b])

    for b in range(NBUF):
        off = base + (NCHUNK - NBUF + b) * CHUNK_T
        pltpu.make_async_copy(out_bufs[b],
                              out_hbm.at[pl.ds(off, CHUNK_T)],
                              osems[b]).wait()


_sc_call = pl.kernel(
    _body,
    out_type=jax.ShapeDtypeStruct((TILES, COLS, TILE_ROWS), jnp.float32),
    mesh=_mesh,
    scratch_types=(
        [pltpu.VMEM((CHUNK_T, COLS - 1, TILE_ROWS), jnp.float32)] * NBUF
        + [pltpu.VMEM((CHUNK_T, COLS, TILE_ROWS), jnp.float32)] * NBUF
        + [pltpu.SemaphoreType.DMA] * (2 * NBUF)
    ),
    compiler_params=pltpu.CompilerParams(needs_layout_passes=False,
                                         use_tc_tiling_on_sc=False),
)


def kernel(input):
    x3 = input.reshape(TILES, TILE_ROWS, COLS).swapaxes(1, 2)
    out3 = _sc_call(x3)
    return out3.swapaxes(1, 2).reshape(ROWS, COLS)


# speedup vs baseline: 1.6660x; 1.0008x over previous
import jax
import jax.numpy as jnp
from jax import lax
from jax.experimental import pallas as pl
from jax.experimental.pallas import tpu as pltpu
from jax.experimental.pallas import tpu_sc as plsc

NC = 2
NS = 16
NW = NC * NS
L = 16

ROWS = 524288
COLS = 8
TILE_ROWS = 128
TILES = ROWS // TILE_ROWS
TPW = TILES // NW
CHUNK_T = 16
NCHUNK = TPW // CHUNK_T
NBUF = 2

_mesh = plsc.VectorSubcoreMesh(core_axis_name="c", subcore_axis_name="s",
                               num_cores=NC, num_subcores=NS)


def _softmax_chunk(in_buf, out_buf):
    zeros = jnp.zeros((L,), jnp.float32)

    @plsc.parallel_loop(0, CHUNK_T * (TILE_ROWS // L), 1, unroll=4)
    def _grp(g):
        t = lax.shift_right_logical(g, 3)
        l0 = lax.shift_left(lax.bitwise_and(g, 7), 4)
        cols = [in_buf[t, c, pl.ds(l0, L)] for c in range(7)]
        m01 = jnp.maximum(cols[0], cols[1])
        m23 = jnp.maximum(cols[2], cols[3])
        m45 = jnp.maximum(cols[4], cols[5])
        m = jnp.maximum(jnp.maximum(m01, m23), jnp.maximum(m45, cols[6]))
        es = [jnp.exp(v - m) for v in cols]
        s01 = es[0] + es[1]
        s23 = es[2] + es[3]
        s45 = es[4] + es[5]
        s = (s01 + s23) + (s45 + es[6])
        inv = jnp.float32(1.0 / 256.0) / s
        for c in range(7):
            out_buf[t, c, pl.ds(l0, L)] = es[c] * inv
        out_buf[t, 7, pl.ds(l0, L)] = zeros


def _body(in_hbm, out_hbm,
          in_v0, in_v1, out_v0, out_v1, isem0, isem1, osem0, osem1):
    in_bufs = (in_v0, in_v1)
    out_bufs = (out_v0, out_v1)
    isems = (isem0, isem1)
    osems = (osem0, osem1)

    wid = lax.axis_index("s") * NC + lax.axis_index("c")
    base = wid * TPW

    for b in range(NBUF):
        pltpu.async_copy(
            in_hbm.at[pl.ds(base + b * CHUNK_T, CHUNK_T), pl.ds(0, 7)],
            in_bufs[b], isems[b])

    @pl.loop(0, NCHUNK, step=NBUF)
    def _outer(g0):
        for b in range(NBUF):
            g = g0 + b
            off = base + g * CHUNK_T
            pltpu.make_async_copy(
                in_hbm.at[pl.ds(off, CHUNK_T), pl.ds(0, 7)],
                in_bufs[b], isems[b]).wait()

            @pl.when(g0 > 0)
            def _():
                pltpu.make_async_copy(
                    out_bufs[b],
                    out_hbm.at[pl.ds(off - NBUF * CHUNK_T, CHUNK_T)],
                    osems[b]).wait()

            _softmax_chunk(in_bufs[b], out_bufs[b])

            pltpu.async_copy(out_bufs[b], out_hbm.at[pl.ds(off, CHUNK_T)],
                             osems[b])

            @pl.when(g + NBUF < NCHUNK)
            def _():
                pltpu.async_copy(
                    in_hbm.at[pl.ds(off + NBUF * CHUNK_T, CHUNK_T),
                              pl.ds(0, 7)],
                    in_bufs[b], isems[b])

    for b in range(NBUF):
        off = base + (NCHUNK - NBUF + b) * CHUNK_T
        pltpu.make_async_copy(out_bufs[b],
                              out_hbm.at[pl.ds(off, CHUNK_T)],
                              osems[b]).wait()


_sc_call = pl.kernel(
    _body,
    out_type=jax.ShapeDtypeStruct((TILES, COLS, TILE_ROWS), jnp.float32),
    mesh=_mesh,
    scratch_types=[
        pltpu.VMEM((CHUNK_T, COLS - 1, TILE_ROWS), jnp.float32),
        pltpu.VMEM((CHUNK_T, COLS - 1, TILE_ROWS), jnp.float32),
        pltpu.VMEM((CHUNK_T, COLS, TILE_ROWS), jnp.float32),
        pltpu.VMEM((CHUNK_T, COLS, TILE_ROWS), jnp.float32),
        pltpu.SemaphoreType.DMA,
        pltpu.SemaphoreType.DMA,
        pltpu.SemaphoreType.DMA,
        pltpu.SemaphoreType.DMA,
    ],
    compiler_params=pltpu.CompilerParams(needs_layout_passes=False,
                                         use_tc_tiling_on_sc=False),
)


def kernel(input):
    x3 = input.reshape(TILES, TILE_ROWS, COLS).swapaxes(1, 2)
    out3 = _sc_call(x3)
    return out3.swapaxes(1, 2).reshape(ROWS, COLS)


# speedup vs baseline: 1.7097x; 1.0263x over previous
import jax
import jax.numpy as jnp
from jax import lax
from jax.experimental import pallas as pl
from jax.experimental.pallas import tpu as pltpu
from jax.experimental.pallas import tpu_sc as plsc

NC = 2
NS = 16
NW = NC * NS
L = 16

ROWS = 524288
COLS = 8
TILE_ROWS = 128
TILES = ROWS // TILE_ROWS
TPW = TILES // NW
CHUNK_T = 16
NCHUNK = TPW // CHUNK_T
NBUF = 2

_mesh = plsc.VectorSubcoreMesh(core_axis_name="c", subcore_axis_name="s",
                               num_cores=NC, num_subcores=NS)


def _softmax_chunk(in_buf, out_buf):
    zeros = jnp.zeros((L,), jnp.float32)

    @plsc.parallel_loop(0, CHUNK_T * (TILE_ROWS // L), 1, unroll=2)
    def _grp(g):
        t = lax.shift_right_logical(g, 3)
        l0 = lax.shift_left(lax.bitwise_and(g, 7), 4)
        cols = [in_buf[t, c, pl.ds(l0, L)] for c in range(7)]
        m01 = jnp.maximum(cols[0], cols[1])
        m23 = jnp.maximum(cols[2], cols[3])
        m45 = jnp.maximum(cols[4], cols[5])
        m = jnp.maximum(jnp.maximum(m01, m23), jnp.maximum(m45, cols[6]))
        es = [jnp.exp(v - m) for v in cols]
        s01 = es[0] + es[1]
        s23 = es[2] + es[3]
        s45 = es[4] + es[5]
        s = (s01 + s23) + (s45 + es[6])
        inv = jnp.float32(1.0 / 256.0) / s
        for c in range(7):
            out_buf[t, c, pl.ds(l0, L)] = es[c] * inv
        out_buf[t, 7, pl.ds(l0, L)] = zeros


def _body(in_hbm, out_hbm,
          in_v0, in_v1, out_v0, out_v1, isem0, isem1, osem0, osem1):
    in_bufs = (in_v0, in_v1)
    out_bufs = (out_v0, out_v1)
    isems = (isem0, isem1)
    osems = (osem0, osem1)

    wid = lax.axis_index("s") * NC + lax.axis_index("c")
    base = wid * TPW

    for b in range(NBUF):
        pltpu.async_copy(
            in_hbm.at[pl.ds(base + b * CHUNK_T, CHUNK_T), pl.ds(0, 7)],
            in_bufs[b], isems[b])

    @pl.loop(0, NCHUNK, step=NBUF)
    def _outer(g0):
        for b in range(NBUF):
            g = g0 + b
            off = base + g * CHUNK_T
            pltpu.make_async_copy(
                in_hbm.at[pl.ds(off, CHUNK_T), pl.ds(0, 7)],
                in_bufs[b], isems[b]).wait()

            @pl.when(g0 > 0)
            def _():
                pltpu.make_async_copy(
                    out_bufs[b],
                    out_hbm.at[pl.ds(off - NBUF * CHUNK_T, CHUNK_T)],
                    osems[b]).wait()

            _softmax_chunk(in_bufs[b], out_bufs[b])

            pltpu.async_copy(out_bufs[b], out_hbm.at[pl.ds(off, CHUNK_T)],
                             osems[b])

            @pl.when(g + NBUF < NCHUNK)
            def _():
                pltpu.async_copy(
                    in_hbm.at[pl.ds(off + NBUF * CHUNK_T, CHUNK_T),
                              pl.ds(0, 7)],
                    in_bufs[b], isems[b])

    for b in range(NBUF):
        off = base + (NCHUNK - NBUF + b) * CHUNK_T
        pltpu.make_async_copy(out_bufs[b],
                              out_hbm.at[pl.ds(off, CHUNK_T)],
                              osems[b]).wait()


_sc_call = pl.kernel(
    _body,
    out_type=jax.ShapeDtypeStruct((TILES, COLS, TILE_ROWS), jnp.float32),
    mesh=_mesh,
    scratch_types=[
        pltpu.VMEM((CHUNK_T, COLS - 1, TILE_ROWS), jnp.float32),
        pltpu.VMEM((CHUNK_T, COLS - 1, TILE_ROWS), jnp.float32),
        pltpu.VMEM((CHUNK_T, COLS, TILE_ROWS), jnp.float32),
        pltpu.VMEM((CHUNK_T, COLS, TILE_ROWS), jnp.float32),
        pltpu.SemaphoreType.DMA,
        pltpu.SemaphoreType.DMA,
        pltpu.SemaphoreType.DMA,
        pltpu.SemaphoreType.DMA,
    ],
    compiler_params=pltpu.CompilerParams(needs_layout_passes=False,
                                         use_tc_tiling_on_sc=False),
)


def kernel(input):
    x3 = input.reshape(TILES, TILE_ROWS, COLS).swapaxes(1, 2)
    out3 = _sc_call(x3)
    return out3.swapaxes(1, 2).reshape(ROWS, COLS)
